# baseline (speedup 1.0000x reference)
import functools

import jax
import jax.numpy as jnp
from jax import lax
from jax.experimental import pallas as pl
from jax.experimental.pallas import tpu as pltpu
from jax.experimental.pallas import tpu_sc as plsc

D_MODEL = 64
CHUNK = 128


@functools.lru_cache(maxsize=None)
def _make_gather(B: int):
    info = plsc.get_sparse_core_info()
    nc, ns = info.num_cores, info.num_subcores
    nw = nc * ns
    assert B % (nw * CHUNK) == 0, (B, nw, CHUNK)
    n_chunks = B // (nw * CHUNK)
    b_per_w = n_chunks * CHUNK

    mesh = plsc.VectorSubcoreMesh(core_axis_name="c", subcore_axis_name="s")

    @functools.partial(
        pl.kernel,
        mesh=mesh,
        out_type=jax.ShapeDtypeStruct((B, D_MODEL), jnp.float32),
        scratch_types=[
            pltpu.VMEM((n_chunks, CHUNK), jnp.int32),
            pltpu.VMEM((CHUNK, D_MODEL), jnp.float32),
            pltpu.SemaphoreType.DMA,
        ],
        compiler_params=pltpu.CompilerParams(use_tc_tiling_on_sc=False),
    )
    def gather_kernel(x_hbm, table_hbm, out_hbm, idx_v, rows_v, sem):
        wid = lax.axis_index("s") * nc + lax.axis_index("c")
        base = wid * b_per_w
        pltpu.sync_copy(x_hbm.at[wid], idx_v)

        def body(c, carry):
            pltpu.async_copy(table_hbm.at[idx_v.at[c]], rows_v, sem).wait()
            pltpu.sync_copy(rows_v, out_hbm.at[pl.ds(base + c * CHUNK, CHUNK)])
            return carry

        lax.fori_loop(0, n_chunks, body, 0)

    return gather_kernel, nw, n_chunks


def kernel(x, table):
    batch, hist = x.shape
    B = batch * hist
    gather, nw, n_chunks = _make_gather(B)
    x_blocked = x.reshape(nw, n_chunks, CHUNK).astype(jnp.int32)
    out = gather(x_blocked, table)
    return out.reshape(batch, hist, D_MODEL)


# speedup vs baseline: 1.1110x; 1.1110x over previous
import functools

import jax
import jax.numpy as jnp
from jax import lax
from jax.experimental import pallas as pl
from jax.experimental.pallas import tpu as pltpu
from jax.experimental.pallas import tpu_sc as plsc

D_MODEL = 64
CHUNK = 128
K = 4


@functools.lru_cache(maxsize=None)
def _make_gather(B: int):
    info = plsc.get_sparse_core_info()
    nc, ns = info.num_cores, info.num_subcores
    nw = nc * ns
    grp = K * CHUNK
    assert B % (nw * grp * 2) == 0, (B, nw, grp)
    n_chunks = B // (nw * CHUNK)
    n_groups = n_chunks // K
    b_per_w = n_chunks * CHUNK

    mesh = plsc.VectorSubcoreMesh(core_axis_name="c", subcore_axis_name="s")

    @functools.partial(
        pl.kernel,
        mesh=mesh,
        out_type=jax.ShapeDtypeStruct((B, D_MODEL), jnp.float32),
        scratch_types=[
            pltpu.VMEM((n_chunks, CHUNK), jnp.int32),
            pltpu.VMEM((grp, D_MODEL), jnp.float32),
            pltpu.VMEM((grp, D_MODEL), jnp.float32),
            pltpu.SemaphoreType.DMA,
            pltpu.SemaphoreType.DMA,
            pltpu.SemaphoreType.DMA,
            pltpu.SemaphoreType.DMA,
        ],
        compiler_params=pltpu.CompilerParams(use_tc_tiling_on_sc=False),
    )
    def gather_kernel(x_hbm, table_hbm, out_hbm, idx_v, rows0, rows1,
                      sem_g0, sem_g1, sem_w0, sem_w1):
        wid = lax.axis_index("s") * nc + lax.axis_index("c")
        base = wid * b_per_w
        pltpu.sync_copy(x_hbm.at[wid], idx_v)

        def fire_gathers(g, rows, sem):
            for j in range(K):
                pltpu.async_copy(table_hbm.at[idx_v.at[g * K + j]],
                                 rows.at[pl.ds(j * CHUNK, CHUNK)], sem)

        def drain(sem, rows):
            pltpu.make_async_copy(table_hbm.at[pl.ds(0, grp)], rows, sem).wait()

        def fire_write(g, rows, sem):
            pltpu.async_copy(rows, out_hbm.at[pl.ds(base + g * grp, grp)], sem)

        def body(t, carry):
            g0 = 2 * t
            g1 = g0 + 1

            @pl.when(t > 0)
            def _():
                drain(sem_w0, rows0)
            fire_gathers(g0, rows0, sem_g0)

            @pl.when(t > 0)
            def _():
                drain(sem_w1, rows1)
            fire_gathers(g1, rows1, sem_g1)

            drain(sem_g0, rows0)
            fire_write(g0, rows0, sem_w0)
            drain(sem_g1, rows1)
            fire_write(g1, rows1, sem_w1)
            return carry

        lax.fori_loop(0, n_groups // 2, body, 0)
        drain(sem_w0, rows0)
        drain(sem_w1, rows1)

    return gather_kernel, nw, n_chunks


def kernel(x, table):
    batch, hist = x.shape
    B = batch * hist
    gather, nw, n_chunks = _make_gather(B)
    x_blocked = x.reshape(nw, n_chunks, CHUNK).astype(jnp.int32)
    out = gather(x_blocked, table)
    return out.reshape(batch, hist, D_MODEL)
